# Initial kernel scaffold; baseline (speedup 1.0000x reference)
import functools

import jax
import jax.numpy as jnp
from jax import lax
from jax.experimental import pallas as pl
from jax.experimental.pallas import tpu as pltpu
from jax.experimental.pallas import tpu_sc as plsc

N = 16384
G = 16
NPC = N // G
K = 30
KP = 32
EP = N * KP
H = 64
OUT_C = 5

_SC_CORES = 2
_SC_SUBCORES = 16
_SC_WORKERS = _SC_CORES * _SC_SUBCORES
_GCHUNK = 128


def _knn_kernel(x_ref, wai_ref, waj_ref, idx_ref, a_ref, b_ref, st_ref,
                d_scr, m_scr):
    pid = pl.program_id(0)
    x = x_ref[0]
    xxt = lax.dot_general(x, x, (((1,), (1,)), ((), ())),
                          preferred_element_type=jnp.float32)
    row_i = lax.broadcasted_iota(jnp.int32, (NPC, NPC), 0)
    col_i = lax.broadcasted_iota(jnp.int32, (NPC, NPC), 1)
    sq_lanes = jnp.sum(jnp.where(row_i == col_i, xxt, 0.0),
                       axis=0, keepdims=True)
    d_scr[...] = sq_lanes - 2.0 * xxt
    m_scr[...] = jnp.zeros((NPC, NPC), jnp.float32)
    idx_ref[0] = jnp.zeros((NPC, KP), jnp.int32)
    kcol = lax.broadcasted_iota(jnp.int32, (NPC, KP), 1)

    def body(k, carry):
        dcur = d_scr[...]
        rowmin = jnp.min(dcur, axis=1, keepdims=True)
        cand = jnp.where(dcur <= rowmin, col_i, NPC)
        sel = jnp.min(cand, axis=1, keepdims=True)
        onehot = col_i == sel
        d_scr[...] = jnp.where(onehot, jnp.float32(1e30), dcur)
        m_scr[...] = m_scr[...] + onehot.astype(jnp.float32)
        idx_ref[0] = jnp.where(kcol == k, sel, idx_ref[0])
        return carry

    lax.fori_loop(0, K, body, 0)
    idx_ref[0] = idx_ref[0] + pid * NPC
    a = jnp.dot(x, wai_ref[...], preferred_element_type=jnp.float32)
    b = jnp.dot(x, waj_ref[...], preferred_element_type=jnp.float32)
    a_ref[0] = a
    b_ref[0] = b
    m = m_scr[...]
    s1 = jnp.dot(m, b, preferred_element_type=jnp.float32)
    s2 = jnp.dot(m, b * b, preferred_element_type=jnp.float32)
    st_ref[0] = jnp.concatenate([
        jnp.sum(a, axis=0, keepdims=True),
        jnp.sum(a * a, axis=0, keepdims=True),
        jnp.sum(s1, axis=0, keepdims=True),
        jnp.sum(s2, axis=0, keepdims=True),
        jnp.sum(a * s1, axis=0, keepdims=True),
        jnp.zeros((3, H), jnp.float32),
    ], axis=0)


def _knn_stage(x, wai, waj):
    c = x.shape[-1]
    return pl.pallas_call(
        _knn_kernel,
        grid=(G,),
        in_specs=[
            pl.BlockSpec((1, NPC, c), lambda i: (i, 0, 0)),
            pl.BlockSpec((c, H), lambda i: (0, 0)),
            pl.BlockSpec((c, H), lambda i: (0, 0)),
        ],
        out_specs=[
            pl.BlockSpec((1, NPC, KP), lambda i: (i, 0, 0)),
            pl.BlockSpec((1, NPC, H), lambda i: (i, 0, 0)),
            pl.BlockSpec((1, NPC, H), lambda i: (i, 0, 0)),
            pl.BlockSpec((1, 8, H), lambda i: (i, 0, 0)),
        ],
        out_shape=[
            jax.ShapeDtypeStruct((G, NPC, KP), jnp.int32),
            jax.ShapeDtypeStruct((G, NPC, H), jnp.float32),
            jax.ShapeDtypeStruct((G, NPC, H), jnp.float32),
            jax.ShapeDtypeStruct((G, 8, H), jnp.float32),
        ],
        scratch_shapes=[
            pltpu.VMEM((NPC, NPC), jnp.float32),
            pltpu.VMEM((NPC, NPC), jnp.float32),
        ],
    )(x, wai, waj)


def _sc_gather(table, idx_flat):
    rows_per_w = EP // _SC_WORKERS
    nchunks = rows_per_w // _GCHUNK
    mesh = plsc.VectorSubcoreMesh(core_axis_name="c", subcore_axis_name="s")

    @functools.partial(
        pl.kernel,
        mesh=mesh,
        out_type=jax.ShapeDtypeStruct((EP, H), jnp.float32),
        scratch_types=[
            pltpu.VMEM((_GCHUNK,), jnp.int32),
            pltpu.VMEM((_GCHUNK, H), jnp.float32),
            pltpu.SemaphoreType.DMA,
        ],
    )
    def gath(table_hbm, idx_hbm, out_hbm, idx_v, rows_v, sem):
        wid = lax.axis_index("s") * _SC_CORES + lax.axis_index("c")
        base = wid * rows_per_w

        def chunk(i, carry):
            off = base + i * _GCHUNK
            pltpu.sync_copy(idx_hbm.at[pl.ds(off, _GCHUNK)], idx_v)
            pltpu.async_copy(table_hbm.at[idx_v], rows_v, sem).wait()
            pltpu.sync_copy(rows_v, out_hbm.at[pl.ds(off, _GCHUNK)])
            return carry

        lax.fori_loop(0, nchunks, chunk, 0)

    return gath(table, idx_flat)


TB = 256


def _edge_kernel(g_ref, a_ref, sc_ref, sh_ref, wb_ref, bb_ref, o_ref):
    gt = g_ref[...].reshape(TB, KP, H)
    a = a_ref[...].reshape(TB, 1, H)
    u = (gt + a) * sc_ref[...].reshape(1, 1, H) + sh_ref[...].reshape(1, 1, H)
    h = jnp.maximum(u, 0.0).reshape(TB * KP, H)
    h2 = jnp.dot(h, wb_ref[...], preferred_element_type=jnp.float32)
    h3 = h2.reshape(TB, KP, H)[:, :K, :]
    o_ref[...] = jnp.max(h3, axis=1) + bb_ref[...]


def _edge_conv_layer(x, Wa, ba, g, be, Wb, bb):
    c = x.shape[-1]
    wai = Wa[:c] - Wa[c:]
    waj = Wa[c:]
    idx, a, b, st = _knn_stage(x, wai, waj)
    stg = jnp.sum(st, axis=0)
    e_cnt = jnp.float32(N * K)
    sum_a, sum_a2, sum_s1, sum_s2, sum_as1 = (stg[0], stg[1], stg[2],
                                              stg[3], stg[4])
    mean_c = (K * sum_a + sum_s1) / e_cnt
    e_c2 = (K * sum_a2 + 2.0 * sum_as1 + sum_s2) / e_cnt
    var = e_c2 - mean_c * mean_c
    invstd = 1.0 / jnp.sqrt(var + 1e-5)
    scale = g * invstd
    shift = be - mean_c * scale

    gath = _sc_gather(b.reshape(N, H), idx.reshape(EP))

    out = pl.pallas_call(
        _edge_kernel,
        grid=(N // TB,),
        in_specs=[
            pl.BlockSpec((TB * KP, H), lambda i: (i, 0)),
            pl.BlockSpec((TB, H), lambda i: (i, 0)),
            pl.BlockSpec((1, H), lambda i: (0, 0)),
            pl.BlockSpec((1, H), lambda i: (0, 0)),
            pl.BlockSpec((H, H), lambda i: (0, 0)),
            pl.BlockSpec((1, H), lambda i: (0, 0)),
        ],
        out_specs=pl.BlockSpec((TB, H), lambda i: (i, 0)),
        out_shape=jax.ShapeDtypeStruct((N, H), jnp.float32),
    )(gath, a.reshape(N, H), scale[None], shift[None], Wb, bb[None])
    return out.reshape(G, NPC, H)


RB = 1024


def _mlp_kernel(x1_ref, x2_ref, x3_ref, w1_ref, b1_ref, w2_ref, b2_ref,
                w3_ref, b3_ref, w4_ref, b4_ref, o_ref):
    h = jnp.concatenate([x1_ref[...], x2_ref[...], x3_ref[...]], axis=1)
    h = jnp.maximum(jnp.dot(h, w1_ref[...],
                            preferred_element_type=jnp.float32)
                    + b1_ref[...], 0.0)
    h = jnp.maximum(jnp.dot(h, w2_ref[...],
                            preferred_element_type=jnp.float32)
                    + b2_ref[...], 0.0)
    h = jnp.maximum(jnp.dot(h, w3_ref[...],
                            preferred_element_type=jnp.float32)
                    + b3_ref[...], 0.0)
    h = jnp.dot(h, w4_ref[...], preferred_element_type=jnp.float32) \
        + b4_ref[...]
    m = jnp.max(h, axis=1, keepdims=True)
    hs = h - m
    o_ref[...] = hs - jnp.log(jnp.sum(jnp.exp(hs), axis=1, keepdims=True))


def _wrap_kernel(s_ref, w1_ref, b1_ref, w2_ref, b2_ref, o_ref):
    h = jnp.maximum(jnp.dot(s_ref[...], w1_ref[...],
                            preferred_element_type=jnp.float32)
                    + b1_ref[...], 0.0)
    o = jnp.dot(h, w2_ref[...], preferred_element_type=jnp.float32) \
        + b2_ref[...]
    m = jnp.max(o, axis=1, keepdims=True)
    os_ = o - m
    o_ref[...] = os_ - jnp.log(jnp.sum(jnp.exp(os_), axis=1, keepdims=True))


def kernel(pos, batch, batch_size, W1a, b1a, g1, be1, W1b, b1b,
           W2a, b2a, g2, be2, W2b, b2b, W3a, b3a, g3, be3, W3b, b3b,
           Wm1, bm1, Wm2, bm2, Wm3, bm3, Wm4, bm4, Ww1, bw1, Ww2, bw2):
    del batch, batch_size
    x0 = pos.reshape(G, NPC, pos.shape[-1])
    x1 = _edge_conv_layer(x0, W1a, b1a, g1, be1, W1b, b1b)
    x2 = _edge_conv_layer(x1, W2a, b2a, g2, be2, W2b, b2b)
    x3 = _edge_conv_layer(x2, W3a, b3a, g3, be3, W3b, b3b)

    seg = pl.pallas_call(
        _mlp_kernel,
        grid=(N // RB,),
        in_specs=[
            pl.BlockSpec((RB, H), lambda i: (i, 0)),
            pl.BlockSpec((RB, H), lambda i: (i, 0)),
            pl.BlockSpec((RB, H), lambda i: (i, 0)),
            pl.BlockSpec(Wm1.shape, lambda i: (0, 0)),
            pl.BlockSpec((1, Wm1.shape[1]), lambda i: (0, 0)),
            pl.BlockSpec(Wm2.shape, lambda i: (0, 0)),
            pl.BlockSpec((1, Wm2.shape[1]), lambda i: (0, 0)),
            pl.BlockSpec(Wm3.shape, lambda i: (0, 0)),
            pl.BlockSpec((1, Wm3.shape[1]), lambda i: (0, 0)),
            pl.BlockSpec(Wm4.shape, lambda i: (0, 0)),
            pl.BlockSpec((1, Wm4.shape[1]), lambda i: (0, 0)),
        ],
        out_specs=pl.BlockSpec((RB, OUT_C), lambda i: (i, 0)),
        out_shape=jax.ShapeDtypeStruct((N, OUT_C), jnp.float32),
    )(x1.reshape(N, H), x2.reshape(N, H), x3.reshape(N, H),
      Wm1, bm1[None], Wm2, bm2[None], Wm3, bm3[None], Wm4, bm4[None])

    segr = seg.reshape(G, NPC * OUT_C)
    out = pl.pallas_call(
        _wrap_kernel,
        grid=(1,),
        in_specs=[
            pl.BlockSpec(segr.shape, lambda i: (0, 0)),
            pl.BlockSpec(Ww1.shape, lambda i: (0, 0)),
            pl.BlockSpec((1, Ww1.shape[1]), lambda i: (0, 0)),
            pl.BlockSpec(Ww2.shape, lambda i: (0, 0)),
            pl.BlockSpec((1, Ww2.shape[1]), lambda i: (0, 0)),
        ],
        out_specs=pl.BlockSpec((G, OUT_C), lambda i: (0, 0)),
        out_shape=jax.ShapeDtypeStruct((G, OUT_C), jnp.float32),
    )(segr, Ww1, bw1[None], Ww2, bw2[None])
    return out


# speedup vs baseline: 8.3383x; 8.3383x over previous
import functools

import jax
import jax.numpy as jnp
from jax import lax
from jax.experimental import pallas as pl
from jax.experimental.pallas import tpu as pltpu
from jax.experimental.pallas import tpu_sc as plsc

N = 16384
G = 16
NPC = N // G
K = 30
KP = 32
EP = N * KP
H = 64
OUT_C = 5

_SC_CORES = 2
_SC_SUBCORES = 16
_SC_WORKERS = _SC_CORES * _SC_SUBCORES
_GCHUNK = 128


def _knn_kernel(x_ref, wai_ref, waj_ref, idx_ref, a_ref, b_ref, st_ref,
                d_scr, m_scr):
    pid = pl.program_id(0)
    x = x_ref[0]
    xxt = lax.dot_general(x, x, (((1,), (1,)), ((), ())),
                          preferred_element_type=jnp.float32,
                          precision=lax.Precision.DEFAULT)
    row_i = lax.broadcasted_iota(jnp.int32, (NPC, NPC), 0)
    col_i = lax.broadcasted_iota(jnp.int32, (NPC, NPC), 1)
    sq_col = jnp.sum(x * x, axis=1, keepdims=True)
    sq_lanes = jnp.sum(jnp.where(row_i == col_i, sq_col, 0.0),
                       axis=0, keepdims=True)
    d_scr[...] = (sq_col + sq_lanes) - 2.0 * xxt
    m_scr[...] = jnp.zeros((NPC, NPC), jnp.float32)
    idx_ref[0] = jnp.zeros((NPC, KP), jnp.int32)
    kcol = lax.broadcasted_iota(jnp.int32, (NPC, KP), 1)

    def body(k, carry):
        dcur = d_scr[...]
        rowmin = jnp.min(dcur, axis=1, keepdims=True)
        cand = jnp.where(dcur <= rowmin, col_i, NPC)
        sel = jnp.min(cand, axis=1, keepdims=True)
        onehot = col_i == sel
        d_scr[...] = jnp.where(onehot, jnp.float32(1e30), dcur)
        m_scr[...] = m_scr[...] + onehot.astype(jnp.float32)
        idx_ref[0] = jnp.where(kcol == k, sel, idx_ref[0])
        return carry

    lax.fori_loop(0, K, body, 0)
    idx_ref[0] = idx_ref[0] + pid * NPC
    a = jnp.dot(x, wai_ref[...], preferred_element_type=jnp.float32)
    b = jnp.dot(x, waj_ref[...], preferred_element_type=jnp.float32)
    a_ref[0] = a
    b_ref[0] = b
    m = m_scr[...]
    s1 = jnp.dot(m, b, preferred_element_type=jnp.float32)
    s2 = jnp.dot(m, b * b, preferred_element_type=jnp.float32)
    st_ref[0] = jnp.concatenate([
        jnp.sum(a, axis=0, keepdims=True),
        jnp.sum(a * a, axis=0, keepdims=True),
        jnp.sum(s1, axis=0, keepdims=True),
        jnp.sum(s2, axis=0, keepdims=True),
        jnp.sum(a * s1, axis=0, keepdims=True),
        jnp.zeros((3, H), jnp.float32),
    ], axis=0)


def _knn_stage(x, wai, waj):
    c = x.shape[-1]
    return pl.pallas_call(
        _knn_kernel,
        grid=(G,),
        in_specs=[
            pl.BlockSpec((1, NPC, c), lambda i: (i, 0, 0)),
            pl.BlockSpec((c, H), lambda i: (0, 0)),
            pl.BlockSpec((c, H), lambda i: (0, 0)),
        ],
        out_specs=[
            pl.BlockSpec((1, NPC, KP), lambda i: (i, 0, 0)),
            pl.BlockSpec((1, NPC, H), lambda i: (i, 0, 0)),
            pl.BlockSpec((1, NPC, H), lambda i: (i, 0, 0)),
            pl.BlockSpec((1, 8, H), lambda i: (i, 0, 0)),
        ],
        out_shape=[
            jax.ShapeDtypeStruct((G, NPC, KP), jnp.int32),
            jax.ShapeDtypeStruct((G, NPC, H), jnp.float32),
            jax.ShapeDtypeStruct((G, NPC, H), jnp.float32),
            jax.ShapeDtypeStruct((G, 8, H), jnp.float32),
        ],
        scratch_shapes=[
            pltpu.VMEM((NPC, NPC), jnp.float32),
            pltpu.VMEM((NPC, NPC), jnp.float32),
        ],
    )(x, wai, waj)


def _sc_gather(table, idx_flat):
    rows_per_w = EP // _SC_WORKERS
    nchunks = rows_per_w // _GCHUNK
    mesh = plsc.VectorSubcoreMesh(core_axis_name="c", subcore_axis_name="s")

    @functools.partial(
        pl.kernel,
        mesh=mesh,
        out_type=jax.ShapeDtypeStruct((EP, H), jnp.float32),
        scratch_types=[
            pltpu.VMEM((_GCHUNK,), jnp.int32),
            pltpu.VMEM((_GCHUNK, H), jnp.float32),
            pltpu.SemaphoreType.DMA,
        ],
        compiler_params=pltpu.CompilerParams(use_tc_tiling_on_sc=False),
    )
    def gath(table_hbm, idx_hbm, out_hbm, idx_v, rows_v, sem):
        wid = lax.axis_index("s") * _SC_CORES + lax.axis_index("c")
        base = wid * rows_per_w

        def chunk(i, carry):
            off = base + i * _GCHUNK
            pltpu.sync_copy(idx_hbm.at[pl.ds(off, _GCHUNK)], idx_v)
            pltpu.async_copy(table_hbm.at[idx_v], rows_v, sem).wait()
            pltpu.sync_copy(rows_v, out_hbm.at[pl.ds(off, _GCHUNK)])
            return carry

        lax.fori_loop(0, nchunks, chunk, 0)

    return gath(table, idx_flat)


TB = 256


def _edge_kernel(g_ref, a_ref, sc_ref, sh_ref, wb_ref, bb_ref, o_ref):
    gt = g_ref[...].reshape(TB, KP, H)
    a = a_ref[...].reshape(TB, 1, H)
    u = (gt + a) * sc_ref[...].reshape(1, 1, H) + sh_ref[...].reshape(1, 1, H)
    h = jnp.maximum(u, 0.0).reshape(TB * KP, H)
    h2 = jnp.dot(h, wb_ref[...], preferred_element_type=jnp.float32)
    h3 = h2.reshape(TB, KP, H)[:, :K, :]
    o_ref[...] = jnp.max(h3, axis=1) + bb_ref[...]


def _edge_conv_layer(x, Wa, ba, g, be, Wb, bb):
    c = x.shape[-1]
    wai = Wa[:c] - Wa[c:]
    waj = Wa[c:]
    idx, a, b, st = _knn_stage(x, wai, waj)
    stg = jnp.sum(st, axis=0)
    e_cnt = jnp.float32(N * K)
    sum_a, sum_a2, sum_s1, sum_s2, sum_as1 = (stg[0], stg[1], stg[2],
                                              stg[3], stg[4])
    mean_c = (K * sum_a + sum_s1) / e_cnt
    e_c2 = (K * sum_a2 + 2.0 * sum_as1 + sum_s2) / e_cnt
    var = e_c2 - mean_c * mean_c
    invstd = 1.0 / jnp.sqrt(var + 1e-5)
    scale = g * invstd
    shift = be - mean_c * scale

    gath = _sc_gather(b.reshape(N, H), idx.reshape(EP))

    out = pl.pallas_call(
        _edge_kernel,
        grid=(N // TB,),
        in_specs=[
            pl.BlockSpec((TB * KP, H), lambda i: (i, 0)),
            pl.BlockSpec((TB, H), lambda i: (i, 0)),
            pl.BlockSpec((1, H), lambda i: (0, 0)),
            pl.BlockSpec((1, H), lambda i: (0, 0)),
            pl.BlockSpec((H, H), lambda i: (0, 0)),
            pl.BlockSpec((1, H), lambda i: (0, 0)),
        ],
        out_specs=pl.BlockSpec((TB, H), lambda i: (i, 0)),
        out_shape=jax.ShapeDtypeStruct((N, H), jnp.float32),
    )(gath, a.reshape(N, H), scale[None], shift[None], Wb, bb[None])
    return out.reshape(G, NPC, H)


RB = 1024


def _mlp_kernel(x1_ref, x2_ref, x3_ref, w1_ref, b1_ref, w2_ref, b2_ref,
                w3_ref, b3_ref, w4_ref, b4_ref, o_ref):
    h = jnp.concatenate([x1_ref[...], x2_ref[...], x3_ref[...]], axis=1)
    h = jnp.maximum(jnp.dot(h, w1_ref[...],
                            preferred_element_type=jnp.float32)
                    + b1_ref[...], 0.0)
    h = jnp.maximum(jnp.dot(h, w2_ref[...],
                            preferred_element_type=jnp.float32)
                    + b2_ref[...], 0.0)
    h = jnp.maximum(jnp.dot(h, w3_ref[...],
                            preferred_element_type=jnp.float32)
                    + b3_ref[...], 0.0)
    h = jnp.dot(h, w4_ref[...], preferred_element_type=jnp.float32) \
        + b4_ref[...]
    m = jnp.max(h, axis=1, keepdims=True)
    hs = h - m
    o_ref[...] = hs - jnp.log(jnp.sum(jnp.exp(hs), axis=1, keepdims=True))


def _wrap_kernel(s_ref, w1_ref, b1_ref, w2_ref, b2_ref, o_ref):
    h = jnp.maximum(jnp.dot(s_ref[...], w1_ref[...],
                            preferred_element_type=jnp.float32)
                    + b1_ref[...], 0.0)
    o = jnp.dot(h, w2_ref[...], preferred_element_type=jnp.float32) \
        + b2_ref[...]
    m = jnp.max(o, axis=1, keepdims=True)
    os_ = o - m
    o_ref[...] = os_ - jnp.log(jnp.sum(jnp.exp(os_), axis=1, keepdims=True))


def kernel(pos, batch, batch_size, W1a, b1a, g1, be1, W1b, b1b,
           W2a, b2a, g2, be2, W2b, b2b, W3a, b3a, g3, be3, W3b, b3b,
           Wm1, bm1, Wm2, bm2, Wm3, bm3, Wm4, bm4, Ww1, bw1, Ww2, bw2):
    del batch, batch_size
    x0 = pos.reshape(G, NPC, pos.shape[-1])
    x1 = _edge_conv_layer(x0, W1a, b1a, g1, be1, W1b, b1b)
    x2 = _edge_conv_layer(x1, W2a, b2a, g2, be2, W2b, b2b)
    x3 = _edge_conv_layer(x2, W3a, b3a, g3, be3, W3b, b3b)

    seg = pl.pallas_call(
        _mlp_kernel,
        grid=(N // RB,),
        in_specs=[
            pl.BlockSpec((RB, H), lambda i: (i, 0)),
            pl.BlockSpec((RB, H), lambda i: (i, 0)),
            pl.BlockSpec((RB, H), lambda i: (i, 0)),
            pl.BlockSpec(Wm1.shape, lambda i: (0, 0)),
            pl.BlockSpec((1, Wm1.shape[1]), lambda i: (0, 0)),
            pl.BlockSpec(Wm2.shape, lambda i: (0, 0)),
            pl.BlockSpec((1, Wm2.shape[1]), lambda i: (0, 0)),
            pl.BlockSpec(Wm3.shape, lambda i: (0, 0)),
            pl.BlockSpec((1, Wm3.shape[1]), lambda i: (0, 0)),
            pl.BlockSpec(Wm4.shape, lambda i: (0, 0)),
            pl.BlockSpec((1, Wm4.shape[1]), lambda i: (0, 0)),
        ],
        out_specs=pl.BlockSpec((RB, OUT_C), lambda i: (i, 0)),
        out_shape=jax.ShapeDtypeStruct((N, OUT_C), jnp.float32),
    )(x1.reshape(N, H), x2.reshape(N, H), x3.reshape(N, H),
      Wm1, bm1[None], Wm2, bm2[None], Wm3, bm3[None], Wm4, bm4[None])

    segr = seg.reshape(G, NPC * OUT_C)
    out = pl.pallas_call(
        _wrap_kernel,
        grid=(1,),
        in_specs=[
            pl.BlockSpec(segr.shape, lambda i: (0, 0)),
            pl.BlockSpec(Ww1.shape, lambda i: (0, 0)),
            pl.BlockSpec((1, Ww1.shape[1]), lambda i: (0, 0)),
            pl.BlockSpec(Ww2.shape, lambda i: (0, 0)),
            pl.BlockSpec((1, Ww2.shape[1]), lambda i: (0, 0)),
        ],
        out_specs=pl.BlockSpec((G, OUT_C), lambda i: (0, 0)),
        out_shape=jax.ShapeDtypeStruct((G, OUT_C), jnp.float32),
    )(segr, Ww1, bw1[None], Ww2, bw2[None])
    return out


# speedup vs baseline: 9.9987x; 1.1991x over previous
import functools

import jax
import jax.numpy as jnp
from jax import lax
from jax.experimental import pallas as pl
from jax.experimental.pallas import tpu as pltpu
from jax.experimental.pallas import tpu_sc as plsc

N = 16384
G = 16
NPC = N // G
K = 30
KP = 32
EP = N * KP
H = 64
OUT_C = 5

_SC_CORES = 2
_SC_SUBCORES = 16
_SC_WORKERS = _SC_CORES * _SC_SUBCORES
_GCHUNK = 128


def _knn_kernel(x_ref, wai_ref, waj_ref, idx_ref, a_ref, b_ref, st_ref,
                d_scr):
    pid = pl.program_id(0)
    x = x_ref[0]
    xxt = lax.dot_general(x, x, (((1,), (1,)), ((), ())),
                          preferred_element_type=jnp.float32,
                          precision=lax.Precision.DEFAULT)
    row_i = lax.broadcasted_iota(jnp.int32, (NPC, NPC), 0)
    col_i = lax.broadcasted_iota(jnp.int32, (NPC, NPC), 1)
    sq_col = jnp.sum(x * x, axis=1, keepdims=True)
    sq_lanes = jnp.sum(jnp.where(row_i == col_i, sq_col, 0.0),
                       axis=0, keepdims=True)
    d_scr[...] = (sq_col + sq_lanes) - 2.0 * xxt
    idx_ref[0] = jnp.zeros((NPC, KP), jnp.int32)
    kcol = lax.broadcasted_iota(jnp.int32, (NPC, KP), 1)

    def body(k, carry):
        dcur = d_scr[...]
        rowmin = jnp.min(dcur, axis=1, keepdims=True)
        cand = jnp.where(dcur <= rowmin, col_i, NPC)
        sel = jnp.min(cand, axis=1, keepdims=True)
        onehot = col_i == sel
        d_scr[...] = jnp.where(onehot, jnp.float32(1e30), dcur)
        idx_ref[0] = jnp.where(kcol == k, sel, idx_ref[0])
        return carry

    lax.fori_loop(0, K, body, 0)
    idx_ref[0] = idx_ref[0] + pid * NPC
    a = jnp.dot(x, wai_ref[...], preferred_element_type=jnp.float32)
    b = jnp.dot(x, waj_ref[...], preferred_element_type=jnp.float32)
    a_ref[0] = a
    b_ref[0] = b
    m = (d_scr[...] >= jnp.float32(1e29)).astype(jnp.float32)
    s1 = jnp.dot(m, b, preferred_element_type=jnp.float32)
    s2 = jnp.dot(m, b * b, preferred_element_type=jnp.float32)
    st_ref[0] = jnp.concatenate([
        jnp.sum(a, axis=0, keepdims=True),
        jnp.sum(a * a, axis=0, keepdims=True),
        jnp.sum(s1, axis=0, keepdims=True),
        jnp.sum(s2, axis=0, keepdims=True),
        jnp.sum(a * s1, axis=0, keepdims=True),
        jnp.zeros((3, H), jnp.float32),
    ], axis=0)


def _knn_stage(x, wai, waj):
    c = x.shape[-1]
    return pl.pallas_call(
        _knn_kernel,
        grid=(G,),
        in_specs=[
            pl.BlockSpec((1, NPC, c), lambda i: (i, 0, 0)),
            pl.BlockSpec((c, H), lambda i: (0, 0)),
            pl.BlockSpec((c, H), lambda i: (0, 0)),
        ],
        out_specs=[
            pl.BlockSpec((1, NPC, KP), lambda i: (i, 0, 0)),
            pl.BlockSpec((1, NPC, H), lambda i: (i, 0, 0)),
            pl.BlockSpec((1, NPC, H), lambda i: (i, 0, 0)),
            pl.BlockSpec((1, 8, H), lambda i: (i, 0, 0)),
        ],
        out_shape=[
            jax.ShapeDtypeStruct((G, NPC, KP), jnp.int32),
            jax.ShapeDtypeStruct((G, NPC, H), jnp.float32),
            jax.ShapeDtypeStruct((G, NPC, H), jnp.float32),
            jax.ShapeDtypeStruct((G, 8, H), jnp.float32),
        ],
        scratch_shapes=[
            pltpu.VMEM((NPC, NPC), jnp.float32),
        ],
        compiler_params=pltpu.CompilerParams(
            dimension_semantics=("parallel",)),
    )(x, wai, waj)


_NBUF = 4


def _sc_gather(table, idx_flat):
    rows_per_w = EP // _SC_WORKERS
    nchunks = rows_per_w // _GCHUNK
    mesh = plsc.VectorSubcoreMesh(core_axis_name="c", subcore_axis_name="s")

    @functools.partial(
        pl.kernel,
        mesh=mesh,
        out_type=jax.ShapeDtypeStruct((EP, H), jnp.float32),
        scratch_types=(
            [pltpu.VMEM((rows_per_w,), jnp.int32)]
            + [pltpu.VMEM((_GCHUNK, H), jnp.float32)] * _NBUF
            + [pltpu.SemaphoreType.DMA] * (2 * _NBUF)
        ),
        compiler_params=pltpu.CompilerParams(use_tc_tiling_on_sc=False),
    )
    def gath(table_hbm, idx_hbm, out_hbm, idx_all, *bufs_sems):
        rows = bufs_sems[:_NBUF]
        gsems = bufs_sems[_NBUF:2 * _NBUF]
        ssems = bufs_sems[2 * _NBUF:]
        wid = lax.axis_index("s") * _SC_CORES + lax.axis_index("c")
        base = wid * rows_per_w
        pltpu.sync_copy(idx_hbm.at[pl.ds(base, rows_per_w)], idx_all)

        def body(i, carry):
            handles = []
            for b in range(_NBUF):
                c = i * _NBUF + b
                off = base + c * _GCHUNK

                @pl.when(i > 0)
                def _(b=b, off=off):
                    pltpu.make_async_copy(
                        rows[b], out_hbm.at[pl.ds(off, _GCHUNK)],
                        ssems[b]).wait()

                handles.append(pltpu.async_copy(
                    table_hbm.at[idx_all.at[pl.ds(c * _GCHUNK, _GCHUNK)]],
                    rows[b], gsems[b]))
            for b in range(_NBUF):
                c = i * _NBUF + b
                off = base + c * _GCHUNK
                handles[b].wait()
                pltpu.async_copy(rows[b], out_hbm.at[pl.ds(off, _GCHUNK)],
                                 ssems[b])
            return carry

        lax.fori_loop(0, nchunks // _NBUF, body, 0)
        for b in range(_NBUF):
            pltpu.make_async_copy(
                rows[b], out_hbm.at[pl.ds(base + b * _GCHUNK, _GCHUNK)],
                ssems[b]).wait()

    return gath(table, idx_flat)


TB = 256


def _edge_kernel(g_ref, a_ref, sc_ref, sh_ref, wb_ref, bb_ref, o_ref):
    gt = g_ref[...].reshape(TB, KP, H)
    a = a_ref[...].reshape(TB, 1, H)
    u = (gt + a) * sc_ref[...].reshape(1, 1, H) + sh_ref[...].reshape(1, 1, H)
    h = jnp.maximum(u, 0.0).reshape(TB * KP, H)
    h2 = jnp.dot(h, wb_ref[...], preferred_element_type=jnp.float32)
    h3 = h2.reshape(TB, KP, H)[:, :K, :]
    o_ref[...] = jnp.max(h3, axis=1) + bb_ref[...]


def _edge_conv_layer(x, Wa, ba, g, be, Wb, bb):
    c = x.shape[-1]
    wai = Wa[:c] - Wa[c:]
    waj = Wa[c:]
    idx, a, b, st = _knn_stage(x, wai, waj)
    stg = jnp.sum(st, axis=0)
    e_cnt = jnp.float32(N * K)
    sum_a, sum_a2, sum_s1, sum_s2, sum_as1 = (stg[0], stg[1], stg[2],
                                              stg[3], stg[4])
    mean_c = (K * sum_a + sum_s1) / e_cnt
    e_c2 = (K * sum_a2 + 2.0 * sum_as1 + sum_s2) / e_cnt
    var = e_c2 - mean_c * mean_c
    invstd = 1.0 / jnp.sqrt(var + 1e-5)
    scale = g * invstd
    shift = be - mean_c * scale

    gath = _sc_gather(b.reshape(N, H), idx.reshape(EP))

    out = pl.pallas_call(
        _edge_kernel,
        grid=(N // TB,),
        in_specs=[
            pl.BlockSpec((TB * KP, H), lambda i: (i, 0)),
            pl.BlockSpec((TB, H), lambda i: (i, 0)),
            pl.BlockSpec((1, H), lambda i: (0, 0)),
            pl.BlockSpec((1, H), lambda i: (0, 0)),
            pl.BlockSpec((H, H), lambda i: (0, 0)),
            pl.BlockSpec((1, H), lambda i: (0, 0)),
        ],
        out_specs=pl.BlockSpec((TB, H), lambda i: (i, 0)),
        out_shape=jax.ShapeDtypeStruct((N, H), jnp.float32),
        compiler_params=pltpu.CompilerParams(
            dimension_semantics=("parallel",)),
    )(gath, a.reshape(N, H), scale[None], shift[None], Wb, bb[None])
    return out.reshape(G, NPC, H)


RB = 1024


def _mlp_kernel(x1_ref, x2_ref, x3_ref, w1_ref, b1_ref, w2_ref, b2_ref,
                w3_ref, b3_ref, w4_ref, b4_ref, o_ref):
    h = jnp.concatenate([x1_ref[...], x2_ref[...], x3_ref[...]], axis=1)
    h = jnp.maximum(jnp.dot(h, w1_ref[...],
                            preferred_element_type=jnp.float32)
                    + b1_ref[...], 0.0)
    h = jnp.maximum(jnp.dot(h, w2_ref[...],
                            preferred_element_type=jnp.float32)
                    + b2_ref[...], 0.0)
    h = jnp.maximum(jnp.dot(h, w3_ref[...],
                            preferred_element_type=jnp.float32)
                    + b3_ref[...], 0.0)
    h = jnp.dot(h, w4_ref[...], preferred_element_type=jnp.float32) \
        + b4_ref[...]
    m = jnp.max(h, axis=1, keepdims=True)
    hs = h - m
    o_ref[...] = hs - jnp.log(jnp.sum(jnp.exp(hs), axis=1, keepdims=True))


def _wrap_kernel(s_ref, w1_ref, b1_ref, w2_ref, b2_ref, o_ref):
    h = jnp.maximum(jnp.dot(s_ref[...], w1_ref[...],
                            preferred_element_type=jnp.float32)
                    + b1_ref[...], 0.0)
    o = jnp.dot(h, w2_ref[...], preferred_element_type=jnp.float32) \
        + b2_ref[...]
    m = jnp.max(o, axis=1, keepdims=True)
    os_ = o - m
    o_ref[...] = os_ - jnp.log(jnp.sum(jnp.exp(os_), axis=1, keepdims=True))


def kernel(pos, batch, batch_size, W1a, b1a, g1, be1, W1b, b1b,
           W2a, b2a, g2, be2, W2b, b2b, W3a, b3a, g3, be3, W3b, b3b,
           Wm1, bm1, Wm2, bm2, Wm3, bm3, Wm4, bm4, Ww1, bw1, Ww2, bw2):
    del batch, batch_size
    x0 = pos.reshape(G, NPC, pos.shape[-1])
    x1 = _edge_conv_layer(x0, W1a, b1a, g1, be1, W1b, b1b)
    x2 = _edge_conv_layer(x1, W2a, b2a, g2, be2, W2b, b2b)
    x3 = _edge_conv_layer(x2, W3a, b3a, g3, be3, W3b, b3b)

    seg = pl.pallas_call(
        _mlp_kernel,
        grid=(N // RB,),
        in_specs=[
            pl.BlockSpec((RB, H), lambda i: (i, 0)),
            pl.BlockSpec((RB, H), lambda i: (i, 0)),
            pl.BlockSpec((RB, H), lambda i: (i, 0)),
            pl.BlockSpec(Wm1.shape, lambda i: (0, 0)),
            pl.BlockSpec((1, Wm1.shape[1]), lambda i: (0, 0)),
            pl.BlockSpec(Wm2.shape, lambda i: (0, 0)),
            pl.BlockSpec((1, Wm2.shape[1]), lambda i: (0, 0)),
            pl.BlockSpec(Wm3.shape, lambda i: (0, 0)),
            pl.BlockSpec((1, Wm3.shape[1]), lambda i: (0, 0)),
            pl.BlockSpec(Wm4.shape, lambda i: (0, 0)),
            pl.BlockSpec((1, Wm4.shape[1]), lambda i: (0, 0)),
        ],
        out_specs=pl.BlockSpec((RB, OUT_C), lambda i: (i, 0)),
        out_shape=jax.ShapeDtypeStruct((N, OUT_C), jnp.float32),
        compiler_params=pltpu.CompilerParams(
            dimension_semantics=("parallel",)),
    )(x1.reshape(N, H), x2.reshape(N, H), x3.reshape(N, H),
      Wm1, bm1[None], Wm2, bm2[None], Wm3, bm3[None], Wm4, bm4[None])

    segr = seg.reshape(G, NPC * OUT_C)
    out = pl.pallas_call(
        _wrap_kernel,
        grid=(1,),
        in_specs=[
            pl.BlockSpec(segr.shape, lambda i: (0, 0)),
            pl.BlockSpec(Ww1.shape, lambda i: (0, 0)),
            pl.BlockSpec((1, Ww1.shape[1]), lambda i: (0, 0)),
            pl.BlockSpec(Ww2.shape, lambda i: (0, 0)),
            pl.BlockSpec((1, Ww2.shape[1]), lambda i: (0, 0)),
        ],
        out_specs=pl.BlockSpec((G, OUT_C), lambda i: (0, 0)),
        out_shape=jax.ShapeDtypeStruct((G, OUT_C), jnp.float32),
    )(segr, Ww1, bw1[None], Ww2, bw2[None])
    return out


# speedup vs baseline: 10.0490x; 1.0050x over previous
import functools

import jax
import jax.numpy as jnp
from jax import lax
from jax.experimental import pallas as pl
from jax.experimental.pallas import tpu as pltpu
from jax.experimental.pallas import tpu_sc as plsc

N = 16384
G = 16
GH = G // 2
NPC = N // G
NH = GH * NPC
K = 30
KP = 32
H = 64
OUT_C = 5

_SC_CORES = 2
_SC_SUBCORES = 16
_SC_WORKERS = _SC_CORES * _SC_SUBCORES
_GCHUNK = 128
_NBUF = 4


def _knn_kernel(x_ref, wai_ref, waj_ref, idx_ref, a_ref, b_ref, st_ref,
                d_scr):
    pid = pl.program_id(0)
    x = x_ref[0]
    xxt = lax.dot_general(x, x, (((1,), (1,)), ((), ())),
                          preferred_element_type=jnp.float32,
                          precision=lax.Precision.DEFAULT)
    row_i = lax.broadcasted_iota(jnp.int32, (NPC, NPC), 0)
    col_i = lax.broadcasted_iota(jnp.int32, (NPC, NPC), 1)
    sq_col = jnp.sum(x * x, axis=1, keepdims=True)
    sq_lanes = jnp.sum(jnp.where(row_i == col_i, sq_col, 0.0),
                       axis=0, keepdims=True)
    d_scr[...] = (sq_col + sq_lanes) - 2.0 * xxt
    idx_ref[0] = jnp.zeros((NPC, KP), jnp.int32)
    kcol = lax.broadcasted_iota(jnp.int32, (NPC, KP), 1)

    def body(k, carry):
        dcur = d_scr[...]
        rowmin = jnp.min(dcur, axis=1, keepdims=True)
        cand = jnp.where(dcur <= rowmin, col_i, NPC)
        sel = jnp.min(cand, axis=1, keepdims=True)
        d_scr[...] = jnp.where(col_i == sel, jnp.float32(1e30), dcur)
        idx_ref[0] = jnp.where(kcol == k, sel, idx_ref[0])
        return carry

    lax.fori_loop(0, K, body, 0)
    idx_ref[0] = idx_ref[0] + pid * NPC
    a = jnp.dot(x, wai_ref[...], preferred_element_type=jnp.float32)
    b = jnp.dot(x, waj_ref[...], preferred_element_type=jnp.float32)
    a_ref[0] = a
    b_ref[0] = b
    m = (d_scr[...] >= jnp.float32(1e29)).astype(jnp.float32)
    s1 = jnp.dot(m, b, preferred_element_type=jnp.float32)
    s2 = jnp.dot(m, b * b, preferred_element_type=jnp.float32)
    st_ref[0] = jnp.concatenate([
        jnp.sum(a, axis=0, keepdims=True),
        jnp.sum(a * a, axis=0, keepdims=True),
        jnp.sum(s1, axis=0, keepdims=True),
        jnp.sum(s2, axis=0, keepdims=True),
        jnp.sum(a * s1, axis=0, keepdims=True),
        jnp.zeros((3, H), jnp.float32),
    ], axis=0)


def _knn_stage(x, wai, waj):
    c = x.shape[-1]
    return pl.pallas_call(
        _knn_kernel,
        grid=(GH,),
        in_specs=[
            pl.BlockSpec((1, NPC, c), lambda i: (i, 0, 0)),
            pl.BlockSpec((c, H), lambda i: (0, 0)),
            pl.BlockSpec((c, H), lambda i: (0, 0)),
        ],
        out_specs=[
            pl.BlockSpec((1, NPC, KP), lambda i: (i, 0, 0)),
            pl.BlockSpec((1, NPC, H), lambda i: (i, 0, 0)),
            pl.BlockSpec((1, NPC, H), lambda i: (i, 0, 0)),
            pl.BlockSpec((1, 8, H), lambda i: (i, 0, 0)),
        ],
        out_shape=[
            jax.ShapeDtypeStruct((GH, NPC, KP), jnp.int32),
            jax.ShapeDtypeStruct((GH, NPC, H), jnp.float32),
            jax.ShapeDtypeStruct((GH, NPC, H), jnp.float32),
            jax.ShapeDtypeStruct((GH, 8, H), jnp.float32),
        ],
        scratch_shapes=[
            pltpu.VMEM((NPC, NPC), jnp.float32),
        ],
        compiler_params=pltpu.CompilerParams(
            dimension_semantics=("parallel",)),
    )(x, wai, waj)


def _sc_gather(table, idx_flat):
    ep = idx_flat.shape[0]
    rows_per_w = ep // _SC_WORKERS
    nchunks = rows_per_w // _GCHUNK
    mesh = plsc.VectorSubcoreMesh(core_axis_name="c", subcore_axis_name="s")

    @functools.partial(
        pl.kernel,
        mesh=mesh,
        out_type=jax.ShapeDtypeStruct((ep, H), jnp.float32),
        scratch_types=(
            [pltpu.VMEM((rows_per_w,), jnp.int32)]
            + [pltpu.VMEM((_GCHUNK, H), jnp.float32)] * _NBUF
            + [pltpu.SemaphoreType.DMA] * (2 * _NBUF)
        ),
        compiler_params=pltpu.CompilerParams(use_tc_tiling_on_sc=False),
    )
    def gath(table_hbm, idx_hbm, out_hbm, idx_all, *bufs_sems):
        rows = bufs_sems[:_NBUF]
        gsems = bufs_sems[_NBUF:2 * _NBUF]
        ssems = bufs_sems[2 * _NBUF:]
        wid = lax.axis_index("s") * _SC_CORES + lax.axis_index("c")
        base = wid * rows_per_w
        pltpu.sync_copy(idx_hbm.at[pl.ds(base, rows_per_w)], idx_all)

        def body(i, carry):
            handles = []
            for b in range(_NBUF):
                c = i * _NBUF + b
                off = base + c * _GCHUNK

                @pl.when(i > 0)
                def _(b=b, off=off):
                    pltpu.make_async_copy(
                        rows[b], out_hbm.at[pl.ds(off, _GCHUNK)],
                        ssems[b]).wait()

                handles.append(pltpu.async_copy(
                    table_hbm.at[idx_all.at[pl.ds(c * _GCHUNK, _GCHUNK)]],
                    rows[b], gsems[b]))
            for b in range(_NBUF):
                c = i * _NBUF + b
                off = base + c * _GCHUNK
                handles[b].wait()
                pltpu.async_copy(rows[b], out_hbm.at[pl.ds(off, _GCHUNK)],
                                 ssems[b])
            return carry

        lax.fori_loop(0, nchunks // _NBUF, body, 0)
        for b in range(_NBUF):
            pltpu.make_async_copy(
                rows[b], out_hbm.at[pl.ds(base + b * _GCHUNK, _GCHUNK)],
                ssems[b]).wait()

    return gath(table, idx_flat)


TB = 256


def _edge_kernel(g_ref, a_ref, sc_ref, sh_ref, wb_ref, bb_ref, o_ref):
    gt = g_ref[...].reshape(TB, KP, H)
    a = a_ref[...].reshape(TB, 1, H)
    u = (gt + a) * sc_ref[...].reshape(1, 1, H) + sh_ref[...].reshape(1, 1, H)
    h = jnp.maximum(u, 0.0).reshape(TB * KP, H)
    h2 = jnp.dot(h, wb_ref[...], preferred_element_type=jnp.float32)
    h3 = h2.reshape(TB, KP, H)[:, :K, :]
    o_ref[...] = jnp.max(h3, axis=1) + bb_ref[...]


def _edge_conv_layer(xs, Wa, ba, g, be, Wb, bb):
    c = xs[0].shape[-1]
    wai = Wa[:c] - Wa[c:]
    waj = Wa[c:]
    knn = [_knn_stage(xh, wai, waj) for xh in xs]
    stg = sum(jnp.sum(st, axis=0) for _, _, _, st in knn)
    e_cnt = jnp.float32(N * K)
    sum_a, sum_a2, sum_s1, sum_s2, sum_as1 = (stg[0], stg[1], stg[2],
                                              stg[3], stg[4])
    mean_c = (K * sum_a + sum_s1) / e_cnt
    e_c2 = (K * sum_a2 + 2.0 * sum_as1 + sum_s2) / e_cnt
    var = e_c2 - mean_c * mean_c
    invstd = 1.0 / jnp.sqrt(var + 1e-5)
    scale = g * invstd
    shift = be - mean_c * scale

    outs = []
    for idx, a, b, _ in knn:
        gath = _sc_gather(b.reshape(NH, H), idx.reshape(NH * KP))
        out = pl.pallas_call(
            _edge_kernel,
            grid=(NH // TB,),
            in_specs=[
                pl.BlockSpec((TB * KP, H), lambda i: (i, 0)),
                pl.BlockSpec((TB, H), lambda i: (i, 0)),
                pl.BlockSpec((1, H), lambda i: (0, 0)),
                pl.BlockSpec((1, H), lambda i: (0, 0)),
                pl.BlockSpec((H, H), lambda i: (0, 0)),
                pl.BlockSpec((1, H), lambda i: (0, 0)),
            ],
            out_specs=pl.BlockSpec((TB, H), lambda i: (i, 0)),
            out_shape=jax.ShapeDtypeStruct((NH, H), jnp.float32),
            compiler_params=pltpu.CompilerParams(
                dimension_semantics=("parallel",)),
        )(gath, a.reshape(NH, H), scale[None], shift[None], Wb, bb[None])
        outs.append(out.reshape(GH, NPC, H))
    return tuple(outs)


RB = 1024


def _mlp_kernel(x1_ref, x2_ref, x3_ref, w1_ref, b1_ref, w2_ref, b2_ref,
                w3_ref, b3_ref, w4_ref, b4_ref, o_ref):
    h = jnp.concatenate([x1_ref[...], x2_ref[...], x3_ref[...]], axis=1)
    h = jnp.maximum(jnp.dot(h, w1_ref[...],
                            preferred_element_type=jnp.float32)
                    + b1_ref[...], 0.0)
    h = jnp.maximum(jnp.dot(h, w2_ref[...],
                            preferred_element_type=jnp.float32)
                    + b2_ref[...], 0.0)
    h = jnp.maximum(jnp.dot(h, w3_ref[...],
                            preferred_element_type=jnp.float32)
                    + b3_ref[...], 0.0)
    h = jnp.dot(h, w4_ref[...], preferred_element_type=jnp.float32) \
        + b4_ref[...]
    m = jnp.max(h, axis=1, keepdims=True)
    hs = h - m
    o_ref[...] = hs - jnp.log(jnp.sum(jnp.exp(hs), axis=1, keepdims=True))


def _wrap_kernel(s_ref, w1_ref, b1_ref, w2_ref, b2_ref, o_ref):
    h = jnp.maximum(jnp.dot(s_ref[...], w1_ref[...],
                            preferred_element_type=jnp.float32)
                    + b1_ref[...], 0.0)
    o = jnp.dot(h, w2_ref[...], preferred_element_type=jnp.float32) \
        + b2_ref[...]
    m = jnp.max(o, axis=1, keepdims=True)
    os_ = o - m
    o_ref[...] = os_ - jnp.log(jnp.sum(jnp.exp(os_), axis=1, keepdims=True))


def kernel(pos, batch, batch_size, W1a, b1a, g1, be1, W1b, b1b,
           W2a, b2a, g2, be2, W2b, b2b, W3a, b3a, g3, be3, W3b, b3b,
           Wm1, bm1, Wm2, bm2, Wm3, bm3, Wm4, bm4, Ww1, bw1, Ww2, bw2):
    del batch, batch_size
    x0 = pos.reshape(G, NPC, pos.shape[-1])
    xs = (x0[:GH], x0[GH:])
    x1 = _edge_conv_layer(xs, W1a, b1a, g1, be1, W1b, b1b)
    x2 = _edge_conv_layer(x1, W2a, b2a, g2, be2, W2b, b2b)
    x3 = _edge_conv_layer(x2, W3a, b3a, g3, be3, W3b, b3b)

    def _full(xh):
        return jnp.concatenate([xh[0].reshape(NH, H), xh[1].reshape(NH, H)],
                               axis=0)

    seg = pl.pallas_call(
        _mlp_kernel,
        grid=(N // RB,),
        in_specs=[
            pl.BlockSpec((RB, H), lambda i: (i, 0)),
            pl.BlockSpec((RB, H), lambda i: (i, 0)),
            pl.BlockSpec((RB, H), lambda i: (i, 0)),
            pl.BlockSpec(Wm1.shape, lambda i: (0, 0)),
            pl.BlockSpec((1, Wm1.shape[1]), lambda i: (0, 0)),
            pl.BlockSpec(Wm2.shape, lambda i: (0, 0)),
            pl.BlockSpec((1, Wm2.shape[1]), lambda i: (0, 0)),
            pl.BlockSpec(Wm3.shape, lambda i: (0, 0)),
            pl.BlockSpec((1, Wm3.shape[1]), lambda i: (0, 0)),
            pl.BlockSpec(Wm4.shape, lambda i: (0, 0)),
            pl.BlockSpec((1, Wm4.shape[1]), lambda i: (0, 0)),
        ],
        out_specs=pl.BlockSpec((RB, OUT_C), lambda i: (i, 0)),
        out_shape=jax.ShapeDtypeStruct((N, OUT_C), jnp.float32),
        compiler_params=pltpu.CompilerParams(
            dimension_semantics=("parallel",)),
    )(_full(x1), _full(x2), _full(x3),
      Wm1, bm1[None], Wm2, bm2[None], Wm3, bm3[None], Wm4, bm4[None])

    segr = seg.reshape(G, NPC * OUT_C)
    out = pl.pallas_call(
        _wrap_kernel,
        grid=(1,),
        in_specs=[
            pl.BlockSpec(segr.shape, lambda i: (0, 0)),
            pl.BlockSpec(Ww1.shape, lambda i: (0, 0)),
            pl.BlockSpec((1, Ww1.shape[1]), lambda i: (0, 0)),
            pl.BlockSpec(Ww2.shape, lambda i: (0, 0)),
            pl.BlockSpec((1, Ww2.shape[1]), lambda i: (0, 0)),
        ],
        out_specs=pl.BlockSpec((G, OUT_C), lambda i: (0, 0)),
        out_shape=jax.ShapeDtypeStruct((G, OUT_C), jnp.float32),
    )(segr, Ww1, bw1[None], Ww2, bw2[None])
    return out
